# Initial kernel scaffold; baseline (speedup 1.0000x reference)
import jax
import jax.numpy as jnp
from jax.experimental import pallas as pl


def kernel(inputs, W, b):
    raise NotImplementedError("write your pallas kernel here")


# speedup vs baseline: 2.3385x; 2.3385x over previous
import jax
import jax.numpy as jnp
from jax import lax
from jax.experimental import pallas as pl
from jax.experimental.pallas import tpu as pltpu

_INPUT_DIM = 768
_NUM_EXPERTS = 64
_N_TOKENS = 32768
_BLK = 2048


def _router_body(x_ref, wt_ref, b_ref, p_ref, i_ref):
    x = x_ref[...]
    wt = wt_ref[...]
    scores = jnp.dot(x, wt, preferred_element_type=jnp.float32)
    scores = scores + b_ref[...]

    iota_e = lax.broadcasted_iota(jnp.int32, scores.shape, 1)
    neg_inf = jnp.float32(-jnp.inf)

    m1 = jnp.max(scores, axis=1, keepdims=True)
    i1 = jnp.min(jnp.where(scores == m1, iota_e, _NUM_EXPERTS), axis=1,
                 keepdims=True)
    masked = jnp.where(iota_e == i1, neg_inf, scores)
    m2 = jnp.max(masked, axis=1, keepdims=True)
    i2 = jnp.min(jnp.where(masked == m2, iota_e, _NUM_EXPERTS), axis=1,
                 keepdims=True)

    e2 = jnp.exp(m2 - m1)
    p1 = 1.0 / (1.0 + e2)
    p2 = e2 * p1

    p_ref[...] = jnp.concatenate([p1, p2], axis=1)
    i_ref[...] = jnp.concatenate([i1, i2], axis=1)


def kernel(inputs, W, b):
    wt = W.T
    b2 = b.reshape(1, _NUM_EXPERTS)
    grid = (_N_TOKENS // _BLK,)
    probs, idx = pl.pallas_call(
        _router_body,
        grid=grid,
        in_specs=[
            pl.BlockSpec((_BLK, _INPUT_DIM), lambda i: (i, 0)),
            pl.BlockSpec((_INPUT_DIM, _NUM_EXPERTS), lambda i: (0, 0)),
            pl.BlockSpec((1, _NUM_EXPERTS), lambda i: (0, 0)),
        ],
        out_specs=[
            pl.BlockSpec((_BLK, 2), lambda i: (i, 0)),
            pl.BlockSpec((_BLK, 2), lambda i: (i, 0)),
        ],
        out_shape=[
            jax.ShapeDtypeStruct((_N_TOKENS, 2), jnp.float32),
            jax.ShapeDtypeStruct((_N_TOKENS, 2), jnp.int32),
        ],
        compiler_params=pltpu.CompilerParams(
            dimension_semantics=("arbitrary",),
        ),
    )(inputs, wt, b2)
    return (probs, idx)


# speedup vs baseline: 2.3801x; 1.0178x over previous
import jax
import jax.numpy as jnp
from jax import lax
from jax.experimental import pallas as pl
from jax.experimental.pallas import tpu as pltpu

_INPUT_DIM = 768
_NUM_EXPERTS = 64
_N_TOKENS = 32768
_BLK = 2048


def _router_body(x_ref, wt_ref, b_ref, iotacol_ref, p_ref, i_ref):
    x = x_ref[...]
    wt = wt_ref[...]
    scores = jnp.dot(x, wt, preferred_element_type=jnp.float32)
    scores = scores + b_ref[...]

    iotacol = iotacol_ref[...]
    neg_inf = jnp.float32(-jnp.inf)

    m1 = jnp.max(scores, axis=1, keepdims=True)
    eq1 = scores == m1
    i1f = jnp.dot(eq1.astype(jnp.float32), iotacol,
                  preferred_element_type=jnp.float32)[:, :1]
    masked = jnp.where(eq1, neg_inf, scores)
    m2 = jnp.max(masked, axis=1, keepdims=True)
    i2f = jnp.dot((masked == m2).astype(jnp.float32), iotacol,
                  preferred_element_type=jnp.float32)[:, :1]
    i1 = i1f.astype(jnp.int32)
    i2 = i2f.astype(jnp.int32)

    e2 = jnp.exp(m2 - m1)
    p1 = 1.0 / (1.0 + e2)
    p2 = e2 * p1

    p_ref[...] = jnp.concatenate([p1, p2], axis=1)
    i_ref[...] = jnp.concatenate([i1, i2], axis=1)


def kernel(inputs, W, b):
    wt = W.T
    b2 = b.reshape(1, _NUM_EXPERTS)
    iotacol = jnp.zeros((_NUM_EXPERTS, 8), jnp.float32).at[:, 0].set(
        jnp.arange(_NUM_EXPERTS, dtype=jnp.float32))
    grid = (_N_TOKENS // _BLK,)
    probs, idx = pl.pallas_call(
        _router_body,
        grid=grid,
        in_specs=[
            pl.BlockSpec((_BLK, _INPUT_DIM), lambda i: (i, 0)),
            pl.BlockSpec((_INPUT_DIM, _NUM_EXPERTS), lambda i: (0, 0)),
            pl.BlockSpec((1, _NUM_EXPERTS), lambda i: (0, 0)),
            pl.BlockSpec((_NUM_EXPERTS, 8), lambda i: (0, 0)),
        ],
        out_specs=[
            pl.BlockSpec((_BLK, 2), lambda i: (i, 0)),
            pl.BlockSpec((_BLK, 2), lambda i: (i, 0)),
        ],
        out_shape=[
            jax.ShapeDtypeStruct((_N_TOKENS, 2), jnp.float32),
            jax.ShapeDtypeStruct((_N_TOKENS, 2), jnp.int32),
        ],
        compiler_params=pltpu.CompilerParams(
            dimension_semantics=("arbitrary",),
        ),
    )(inputs, wt, b2, iotacol)
    return (probs, idx)


# speedup vs baseline: 2.5326x; 1.0641x over previous
import jax
import jax.numpy as jnp
from jax import lax
from jax.experimental import pallas as pl
from jax.experimental.pallas import tpu as pltpu

_INPUT_DIM = 768
_NUM_EXPERTS = 64
_N_TOKENS = 32768
_BLK = 4096


def _router_body(x_ref, wt_ref, b_ref, iotacol_ref, p_ref, i_ref):
    x = x_ref[...]
    wt = wt_ref[...]
    scores = jnp.dot(x, wt, preferred_element_type=jnp.float32)
    scores = scores + b_ref[...]

    iotacol = iotacol_ref[...]
    neg_inf = jnp.float32(-jnp.inf)

    m1 = jnp.max(scores, axis=1, keepdims=True)
    eq1 = scores == m1
    i1f = jnp.dot(eq1.astype(jnp.float32), iotacol,
                  preferred_element_type=jnp.float32)[:, :1]
    masked = jnp.where(eq1, neg_inf, scores)
    m2 = jnp.max(masked, axis=1, keepdims=True)
    i2f = jnp.dot((masked == m2).astype(jnp.float32), iotacol,
                  preferred_element_type=jnp.float32)[:, :1]
    i1 = i1f.astype(jnp.int32)
    i2 = i2f.astype(jnp.int32)

    e2 = jnp.exp(m2 - m1)
    p1 = 1.0 / (1.0 + e2)
    p2 = e2 * p1

    p_ref[...] = jnp.concatenate([p1, p2], axis=1)
    i_ref[...] = jnp.concatenate([i1, i2], axis=1)


def kernel(inputs, W, b):
    wt = W.T
    b2 = b.reshape(1, _NUM_EXPERTS)
    iotacol = jnp.zeros((_NUM_EXPERTS, 8), jnp.float32).at[:, 0].set(
        jnp.arange(_NUM_EXPERTS, dtype=jnp.float32))
    grid = (_N_TOKENS // _BLK,)
    probs, idx = pl.pallas_call(
        _router_body,
        grid=grid,
        in_specs=[
            pl.BlockSpec((_BLK, _INPUT_DIM), lambda i: (i, 0)),
            pl.BlockSpec((_INPUT_DIM, _NUM_EXPERTS), lambda i: (0, 0)),
            pl.BlockSpec((1, _NUM_EXPERTS), lambda i: (0, 0)),
            pl.BlockSpec((_NUM_EXPERTS, 8), lambda i: (0, 0)),
        ],
        out_specs=[
            pl.BlockSpec((_BLK, 2), lambda i: (i, 0)),
            pl.BlockSpec((_BLK, 2), lambda i: (i, 0)),
        ],
        out_shape=[
            jax.ShapeDtypeStruct((_N_TOKENS, 2), jnp.float32),
            jax.ShapeDtypeStruct((_N_TOKENS, 2), jnp.int32),
        ],
        compiler_params=pltpu.CompilerParams(
            dimension_semantics=("arbitrary",),
        ),
    )(inputs, wt, b2, iotacol)
    return (probs, idx)
